# baseline (device time: 13701 ns/iter reference)
import jax
import jax.numpy as jnp
from jax import lax
from jax.experimental import pallas as pl
from jax.experimental.pallas import tpu as pltpu

N_DEV = 16
BLK = 128
N_CHUNK = 16
N_OGRP = 16

_DEV_ID_TYPE = getattr(pltpu, "DeviceIdType", None) or pl.DeviceIdType


def kernel(x):
    m, n = x.shape
    n_blk = m // BLK
    crows = m // N_CHUNK
    orows = m // N_OGRP

    def body(
        x_ref,
        out_ref,
        stage_ref,
        outv_ref,
        total_ref,
        comm_ref,
        in_sems,
        out_sems,
        send_sems,
        recv_sems,
    ):
        me = lax.axis_index("i")

        barrier_sem = pltpu.get_barrier_semaphore()
        for p in range(N_DEV):

            @pl.when(me != p)
            def _signal(p=p):
                pl.semaphore_signal(
                    barrier_sem,
                    inc=1,
                    device_id=(p,),
                    device_id_type=_DEV_ID_TYPE.MESH,
                )

        def in_copy(k):
            return pltpu.make_async_copy(
                x_ref.at[pl.ds(k * crows, crows), :],
                stage_ref.at[pl.ds(k * crows, crows), :],
                in_sems.at[k],
            )

        for k in range(N_CHUNK):
            in_copy(k).start()

        r = lax.broadcasted_iota(jnp.int32, (BLK, BLK), 0)
        c = lax.broadcasted_iota(jnp.int32, (BLK, BLK), 1)
        tri = (r >= c).astype(jnp.bfloat16)

        def cumsum_block(g, off):
            blk = stage_ref[pl.ds(g * BLK, BLK), :].astype(jnp.bfloat16)
            cs = jax.lax.dot(tri, blk, preferred_element_type=jnp.float32)
            outv_ref[pl.ds(g * BLK, BLK), :] = (cs + off).astype(jnp.bfloat16)
            return off + cs[BLK - 1 : BLK, :]

        off = jnp.zeros((1, n), jnp.float32)
        tot = jnp.zeros((1, n), jnp.float32)
        bpc = crows // BLK
        for k in range(N_CHUNK - 1):
            in_copy(k).wait()
            tot = tot + jnp.sum(
                stage_ref[pl.ds(k * crows, crows), :], axis=0, keepdims=True
            )
            for b in range(bpc):
                off = cumsum_block(k * bpc + b, off)

        last = N_CHUNK - 1
        in_copy(last).wait()
        tot = tot + jnp.sum(
            stage_ref[pl.ds(last * crows, crows), :], axis=0, keepdims=True
        )
        total_ref[0, :] = tot[0, :]

        pl.semaphore_wait(barrier_sem, N_DEV - 1)

        for j in range(1, N_DEV):

            @pl.when(me < j)
            def _send(j=j):
                rdma = pltpu.make_async_remote_copy(
                    src_ref=total_ref.at[0],
                    dst_ref=comm_ref.at[me],
                    send_sem=send_sems.at[j],
                    recv_sem=recv_sems.at[me],
                    device_id=(j,),
                    device_id_type=_DEV_ID_TYPE.MESH,
                )
                rdma.start()

        for b in range(bpc):
            off = cumsum_block(last * bpc + b, off)

        for k in range(N_DEV - 1):

            @pl.when(k < me)
            def _recv(k=k):
                rdma = pltpu.make_async_remote_copy(
                    src_ref=total_ref.at[0],
                    dst_ref=comm_ref.at[k],
                    send_sem=send_sems.at[k],
                    recv_sem=recv_sems.at[k],
                    device_id=(0,),
                    device_id_type=_DEV_ID_TYPE.MESH,
                )
                rdma.wait_recv()

        row_ids = lax.broadcasted_iota(jnp.int32, (N_DEV, n), 0)
        comm = comm_ref[:, :]
        offset16 = jnp.sum(
            jnp.where(row_ids < me, comm, jnp.zeros_like(comm)),
            axis=0,
            keepdims=True,
        ).astype(jnp.bfloat16)

        def out_copy(grp):
            return pltpu.make_async_copy(
                outv_ref.at[pl.ds(grp * orows, orows), :],
                out_ref.at[pl.ds(grp * orows, orows), :],
                out_sems.at[grp],
            )

        for grp in range(N_OGRP):
            rows = pl.ds(grp * orows, orows)
            outv_ref[rows, :] = outv_ref[rows, :] + offset16
            out_copy(grp).start()
        for grp in range(N_OGRP):
            out_copy(grp).wait()

        for j in range(1, N_DEV):

            @pl.when(me < j)
            def _wait_send(j=j):
                rdma = pltpu.make_async_remote_copy(
                    src_ref=total_ref.at[0],
                    dst_ref=comm_ref.at[me],
                    send_sem=send_sems.at[j],
                    recv_sem=recv_sems.at[me],
                    device_id=(j,),
                    device_id_type=_DEV_ID_TYPE.MESH,
                )
                rdma.wait_send()

    return pl.pallas_call(
        body,
        out_shape=jax.ShapeDtypeStruct((m, n), jnp.bfloat16),
        in_specs=[pl.BlockSpec(memory_space=pl.ANY)],
        out_specs=pl.BlockSpec(memory_space=pl.ANY),
        scratch_shapes=[
            pltpu.VMEM((m, n), jnp.float32),
            pltpu.VMEM((m, n), jnp.bfloat16),
            pltpu.VMEM((1, n), jnp.float32),
            pltpu.VMEM((N_DEV, n), jnp.float32),
            pltpu.SemaphoreType.DMA((N_CHUNK,)),
            pltpu.SemaphoreType.DMA((N_OGRP,)),
            pltpu.SemaphoreType.DMA((N_DEV,)),
            pltpu.SemaphoreType.DMA((N_DEV,)),
        ],
        compiler_params=pltpu.CompilerParams(collective_id=0),
    )(x)


# device time: 13323 ns/iter; 1.0284x vs baseline; 1.0284x over previous
import jax
import jax.numpy as jnp
from jax import lax
from jax.experimental import pallas as pl
from jax.experimental.pallas import tpu as pltpu

N_DEV = 16
BLK = 128
N_CHUNK = 8
N_OGRP = 8

_DEV_ID_TYPE = getattr(pltpu, "DeviceIdType", None) or pl.DeviceIdType


def kernel(x):
    m, n = x.shape
    n_blk = m // BLK
    crows = m // N_CHUNK
    orows = m // N_OGRP

    def body(
        x_ref,
        out_ref,
        stage_ref,
        outv_ref,
        total_ref,
        comm_ref,
        in_sems,
        out_sems,
        send_sems,
        recv_sems,
    ):
        me = lax.axis_index("i")

        barrier_sem = pltpu.get_barrier_semaphore()
        for p in range(N_DEV):

            @pl.when(me != p)
            def _signal(p=p):
                pl.semaphore_signal(
                    barrier_sem,
                    inc=1,
                    device_id=(p,),
                    device_id_type=_DEV_ID_TYPE.MESH,
                )

        def in_copy(k):
            return pltpu.make_async_copy(
                x_ref.at[pl.ds(k * crows, crows), :],
                stage_ref.at[pl.ds(k * crows, crows), :],
                in_sems.at[k],
            )

        for k in range(N_CHUNK):
            in_copy(k).start()

        r = lax.broadcasted_iota(jnp.int32, (BLK, BLK), 0)
        c = lax.broadcasted_iota(jnp.int32, (BLK, BLK), 1)
        tri = (r >= c).astype(jnp.bfloat16)

        def cumsum_block(g, off):
            blk = stage_ref[pl.ds(g * BLK, BLK), :].astype(jnp.bfloat16)
            cs = jax.lax.dot(tri, blk, preferred_element_type=jnp.float32)
            outv_ref[pl.ds(g * BLK, BLK), :] = (cs + off).astype(jnp.bfloat16)
            return off + cs[BLK - 1 : BLK, :]

        off = jnp.zeros((1, n), jnp.float32)
        tot = jnp.zeros((1, n), jnp.float32)
        bpc = crows // BLK
        for k in range(N_CHUNK - 1):
            in_copy(k).wait()
            tot = tot + jnp.sum(
                stage_ref[pl.ds(k * crows, crows), :], axis=0, keepdims=True
            )
            for b in range(bpc):
                off = cumsum_block(k * bpc + b, off)

        last = N_CHUNK - 1
        in_copy(last).wait()
        tot = tot + jnp.sum(
            stage_ref[pl.ds(last * crows, crows), :], axis=0, keepdims=True
        )
        total_ref[0, :] = tot[0, :]

        pl.semaphore_wait(barrier_sem, N_DEV - 1)

        for j in range(1, N_DEV):

            @pl.when(me < j)
            def _send(j=j):
                rdma = pltpu.make_async_remote_copy(
                    src_ref=total_ref.at[0],
                    dst_ref=comm_ref.at[me],
                    send_sem=send_sems.at[j],
                    recv_sem=recv_sems.at[me],
                    device_id=(j,),
                    device_id_type=_DEV_ID_TYPE.MESH,
                )
                rdma.start()

        for b in range(bpc):
            off = cumsum_block(last * bpc + b, off)

        for k in range(N_DEV - 1):

            @pl.when(k < me)
            def _recv(k=k):
                rdma = pltpu.make_async_remote_copy(
                    src_ref=total_ref.at[0],
                    dst_ref=comm_ref.at[k],
                    send_sem=send_sems.at[k],
                    recv_sem=recv_sems.at[k],
                    device_id=(0,),
                    device_id_type=_DEV_ID_TYPE.MESH,
                )
                rdma.wait_recv()

        row_ids = lax.broadcasted_iota(jnp.int32, (N_DEV, n), 0)
        comm = comm_ref[:, :]
        offset16 = jnp.sum(
            jnp.where(row_ids < me, comm, jnp.zeros_like(comm)),
            axis=0,
            keepdims=True,
        ).astype(jnp.bfloat16)

        def out_copy(grp):
            return pltpu.make_async_copy(
                outv_ref.at[pl.ds(grp * orows, orows), :],
                out_ref.at[pl.ds(grp * orows, orows), :],
                out_sems.at[grp],
            )

        for grp in range(N_OGRP):
            rows = pl.ds(grp * orows, orows)
            outv_ref[rows, :] = outv_ref[rows, :] + offset16
            out_copy(grp).start()
        for grp in range(N_OGRP):
            out_copy(grp).wait()

        for j in range(1, N_DEV):

            @pl.when(me < j)
            def _wait_send(j=j):
                rdma = pltpu.make_async_remote_copy(
                    src_ref=total_ref.at[0],
                    dst_ref=comm_ref.at[me],
                    send_sem=send_sems.at[j],
                    recv_sem=recv_sems.at[me],
                    device_id=(j,),
                    device_id_type=_DEV_ID_TYPE.MESH,
                )
                rdma.wait_send()

    return pl.pallas_call(
        body,
        out_shape=jax.ShapeDtypeStruct((m, n), jnp.bfloat16),
        in_specs=[pl.BlockSpec(memory_space=pl.ANY)],
        out_specs=pl.BlockSpec(memory_space=pl.ANY),
        scratch_shapes=[
            pltpu.VMEM((m, n), jnp.float32),
            pltpu.VMEM((m, n), jnp.bfloat16),
            pltpu.VMEM((1, n), jnp.float32),
            pltpu.VMEM((N_DEV, n), jnp.float32),
            pltpu.SemaphoreType.DMA((N_CHUNK,)),
            pltpu.SemaphoreType.DMA((N_OGRP,)),
            pltpu.SemaphoreType.DMA((N_DEV,)),
            pltpu.SemaphoreType.DMA((N_DEV,)),
        ],
        compiler_params=pltpu.CompilerParams(collective_id=0),
    )(x)
